# Initial kernel scaffold; baseline (speedup 1.0000x reference)
import jax
import jax.numpy as jnp
from jax.experimental import pallas as pl


def kernel(x, edge_index, W, b):
    raise NotImplementedError("write your pallas kernel here")


# speedup vs baseline: 11.6247x; 11.6247x over previous
import functools

import jax
import jax.numpy as jnp
from jax import lax
from jax.experimental import pallas as pl
from jax.experimental.pallas import tpu as pltpu
from jax.experimental.pallas import tpu_sc as plsc

N_NODES = 10000
N_EDGES = 320000
D = 128

NC = 2
NS = 16
NW = NC * NS
K = 128

NPAD = 10240
EPAD = 327680
EPW = EPAD // NW
CHUNKS = EPW // K
RPW = NPAD // NS


def _hist_body(colp_hbm, hist_hbm, cidx_v, ones_v, zrow_v, hist_sh):
    cid = lax.axis_index("c")
    sid = lax.axis_index("s")
    for j in range(8):
        zrow_v[pl.ds(j * 16, 16)] = jnp.zeros((16,), jnp.float32)
        ones_v[pl.ds(j * 16, 16)] = jnp.ones((16,), jnp.float32)
    slab = sid * RPW
    for k in range(RPW // K):
        pltpu.sync_copy(zrow_v, hist_sh.at[pl.ds(slab + k * K, K)])
    plsc.subcore_barrier()

    wid = sid * NC + cid
    ebase = wid * EPW

    def chunk(i, carry):
        pltpu.sync_copy(colp_hbm.at[pl.ds(ebase + i * K, K)], cidx_v)
        pltpu.sync_copy(ones_v, hist_sh.at[cidx_v], add=True)
        return carry

    lax.fori_loop(0, CHUNKS, chunk, 0)
    plsc.subcore_barrier()
    pltpu.sync_copy(hist_sh.at[pl.ds(slab, RPW)],
                    hist_hbm.at[cid, pl.ds(slab, RPW)])


_sc_hist = pl.kernel(
    _hist_body,
    out_type=jax.ShapeDtypeStruct((NC, NPAD), jnp.float32),
    mesh=plsc.VectorSubcoreMesh(core_axis_name="c", subcore_axis_name="s"),
    scratch_types=[
        pltpu.VMEM((K,), jnp.int32),
        pltpu.VMEM((K,), jnp.float32),
        pltpu.VMEM((K,), jnp.float32),
        pltpu.VMEM_SHARED((NPAD,), jnp.float32),
    ],
)


def _scale_body(x_ref, w_ref, hist_ref, g_ref, dinv_ref):
    deg = hist_ref[0, :] + hist_ref[1, :] + 1.0
    dinv = lax.rsqrt(deg)
    h = jnp.dot(x_ref[...], w_ref[...], preferred_element_type=jnp.float32)
    g_ref[...] = h * dinv[:, None]
    dinv_ref[...] = dinv


_BM = 256


def _tc_scale(xp, W, hist):
    return pl.pallas_call(
        _scale_body,
        grid=(NPAD // _BM,),
        in_specs=[
            pl.BlockSpec((_BM, D), lambda i: (i, 0)),
            pl.BlockSpec((D, D), lambda i: (0, 0)),
            pl.BlockSpec((NC, _BM), lambda i: (0, i)),
        ],
        out_specs=[
            pl.BlockSpec((_BM, D), lambda i: (i, 0)),
            pl.BlockSpec((_BM,), lambda i: (i,)),
        ],
        out_shape=[
            jax.ShapeDtypeStruct((NPAD, D), jnp.float32),
            jax.ShapeDtypeStruct((NPAD,), jnp.float32),
        ],
    )(xp, W, hist)


def _scatter_body(rowp_hbm, colp_hbm, g_hbm, acc_hbm,
                  ridx_v, cidx_v, rows_v, acc_sh):
    cid = lax.axis_index("c")
    sid = lax.axis_index("s")
    slab = sid * RPW
    pltpu.sync_copy(g_hbm.at[pl.ds(slab, RPW)], acc_sh.at[pl.ds(slab, RPW)])
    plsc.subcore_barrier()

    wid = sid * NC + cid
    ebase = wid * EPW

    def chunk(i, carry):
        e0 = ebase + i * K
        pltpu.sync_copy(rowp_hbm.at[pl.ds(e0, K)], ridx_v)
        pltpu.sync_copy(colp_hbm.at[pl.ds(e0, K)], cidx_v)
        pltpu.sync_copy(g_hbm.at[ridx_v], rows_v)
        pltpu.sync_copy(rows_v, acc_sh.at[cidx_v], add=True)
        return carry

    lax.fori_loop(0, CHUNKS, chunk, 0)
    plsc.subcore_barrier()
    pltpu.sync_copy(acc_sh.at[pl.ds(slab, RPW)],
                    acc_hbm.at[cid, pl.ds(slab, RPW)])


_sc_scatter = pl.kernel(
    _scatter_body,
    out_type=jax.ShapeDtypeStruct((NC, NPAD, D), jnp.float32),
    mesh=plsc.VectorSubcoreMesh(core_axis_name="c", subcore_axis_name="s"),
    scratch_types=[
        pltpu.VMEM((K,), jnp.int32),
        pltpu.VMEM((K,), jnp.int32),
        pltpu.VMEM((K, D), jnp.float32),
        pltpu.VMEM_SHARED((NPAD, D), jnp.float32),
    ],
)


def _combine_body(acc_ref, g_ref, dinv_ref, b_ref, out_ref):
    t = acc_ref[0] + acc_ref[1] - g_ref[...]
    t = t * dinv_ref[...][:, None] + b_ref[...][None, :]
    m = jnp.max(t, axis=1, keepdims=True)
    lse = jnp.log(jnp.sum(jnp.exp(t - m), axis=1, keepdims=True)) + m
    out_ref[...] = t - lse


def _tc_combine(acc, g, dinv, b):
    return pl.pallas_call(
        _combine_body,
        grid=(NPAD // _BM,),
        in_specs=[
            pl.BlockSpec((NC, _BM, D), lambda i: (0, i, 0)),
            pl.BlockSpec((_BM, D), lambda i: (i, 0)),
            pl.BlockSpec((_BM,), lambda i: (i,)),
            pl.BlockSpec((D,), lambda i: (0,)),
        ],
        out_specs=pl.BlockSpec((_BM, D), lambda i: (i, 0)),
        out_shape=jax.ShapeDtypeStruct((NPAD, D), jnp.float32),
    )(acc, g, dinv, b)


def kernel(x, edge_index, W, b):
    row = edge_index[0]
    col = edge_index[1]
    pad = jnp.full((EPAD - N_EDGES,), N_NODES, dtype=jnp.int32)
    rowp = jnp.concatenate([row, pad])
    colp = jnp.concatenate([col, pad])
    xp = jnp.pad(x, ((0, NPAD - N_NODES), (0, 0)))

    hist = _sc_hist(colp)
    g, dinv = _tc_scale(xp, W, hist)
    acc = _sc_scatter(rowp, colp, g)
    out = _tc_combine(acc, g, dinv, b)
    return out[:N_NODES]
